# Initial kernel scaffold; baseline (speedup 1.0000x reference)
import jax
import jax.numpy as jnp
from jax.experimental import pallas as pl


def kernel(node_attr, edge_index, edge_attr, edge_sh, W1, b1, W2, b2):
    raise NotImplementedError("write your pallas kernel here")


# speedup vs baseline: 4.5856x; 4.5856x over previous
import functools

import jax
import jax.numpy as jnp
import numpy as np
from jax import lax
from jax.experimental import pallas as pl
from jax.experimental.pallas import tpu as pltpu
from jax.experimental.pallas import tpu_sc as plsc

N_NODES = 10000
N_EDGES = 160000
MUL = 32

NW = 32
CHUNK = 128
CHUNKS = 40
EP = NW * CHUNKS * CHUNK
SN = 10112
STRIPE = SN // 16

ISQ3 = float(1.0 / np.sqrt(3.0))
A0 = float(np.sqrt(1.0 / 64.0))
A1 = float(np.sqrt(3.0 / 64.0))

@functools.cache
def _mesh():
    return plsc.VectorSubcoreMesh(core_axis_name="c", subcore_axis_name="s",
                                  num_cores=2, num_subcores=16)

_PERM = np.concatenate([np.arange(32)] + [32 + 3 * np.arange(32) + m for m in range(3)])


def _gather_body(nap_hbm, idx_hbm, out_hbm, idx_v, rows_v, sem):
    c = lax.axis_index("c")
    s = lax.axis_index("s")
    wid = s * 2 + c

    def body(j, carry):
        r = wid * CHUNKS + j
        pltpu.sync_copy(idx_hbm.at[r], idx_v)
        pltpu.async_copy(nap_hbm.at[idx_v], rows_v, sem).wait()
        pltpu.sync_copy(rows_v, out_hbm.at[pl.ds(r * CHUNK, CHUNK)])
        return carry

    lax.fori_loop(0, CHUNKS, body, 0)


@functools.cache
def _sc_gather():
    return pl.kernel(
        _gather_body,
        out_type=jax.ShapeDtypeStruct((EP, 128), jnp.float32),
        mesh=_mesh(),
        scratch_types=[
            pltpu.VMEM((CHUNK,), jnp.int32),
            pltpu.VMEM((CHUNK, 128), jnp.float32),
            pltpu.SemaphoreType.DMA,
        ],
    )


def _scatter_body(tp_hbm, src_hbm, zs_hbm, ones_hbm, sums_out, cnts_out,
                  idx_cur, rows_v, ones_v, acc_sh):
    c = lax.axis_index("c")
    s = lax.axis_index("s")
    wid = s * 2 + c
    pltpu.sync_copy(ones_hbm, ones_v)

    def phase(out_ref, scatter_tp):
        pltpu.sync_copy(zs_hbm.at[pl.ds(s * STRIPE, STRIPE)],
                        acc_sh.at[pl.ds(s * STRIPE, STRIPE)])
        plsc.subcore_barrier()

        def body(j, carry):
            r = wid * CHUNKS + j
            pltpu.sync_copy(src_hbm.at[r], idx_cur)
            if scatter_tp:
                pltpu.sync_copy(tp_hbm.at[pl.ds(r * CHUNK, CHUNK)], rows_v)
                pltpu.sync_copy(rows_v, acc_sh.at[idx_cur], add=True)
            else:
                pltpu.sync_copy(ones_v, acc_sh.at[idx_cur], add=True)
            return carry

        lax.fori_loop(0, CHUNKS, body, 0)
        plsc.subcore_barrier()
        pltpu.sync_copy(acc_sh.at[pl.ds(s * STRIPE, STRIPE)],
                        out_ref.at[c, pl.ds(s * STRIPE, STRIPE)])
        plsc.subcore_barrier()

    phase(sums_out, True)
    phase(cnts_out, False)


@functools.cache
def _sc_scatter():
    return pl.kernel(
        _scatter_body,
        out_type=(
            jax.ShapeDtypeStruct((2, SN, 128), jnp.float32),
            jax.ShapeDtypeStruct((2, SN, 128), jnp.float32),
        ),
        mesh=_mesh(),
        scratch_types=[
            pltpu.VMEM((CHUNK,), jnp.int32),
            pltpu.VMEM((CHUNK, 128), jnp.float32),
            pltpu.VMEM((CHUNK, 128), jnp.float32),
            pltpu.VMEM_SHARED((SN, 128), jnp.float32),
        ],
    )


TCB = 256


def _expand(a, b):
    return jnp.broadcast_to(a[:, None, :], (32, 32, b)).reshape(32 * 32, b)


def _fold(p, b):
    for h in (512, 256, 128, 64, 32):
        p = p[:h] + p[h:2 * h]
    return p


def _tc_compute_body(x_ref, ea_ref, esh_ref, w1t_ref, w2t_ref, tp_ref):
    b = TCB
    xt = x_ref[...].T
    ht = jnp.maximum(jnp.dot(w1t_ref[...], ea_ref[...],
                             preferred_element_type=jnp.float32), 0.0)
    hta = jnp.concatenate([ht, jnp.ones((1, b), jnp.float32)], axis=0)
    wt = jnp.dot(w2t_ref[...], hta.astype(jnp.bfloat16),
                 preferred_element_type=jnp.float32)

    s1 = xt[0:32]
    v1 = [xt[32 + 32 * m:64 + 32 * m] for m in range(3)]
    s2 = esh_ref[0:1]
    v2 = [esh_ref[1 + m:2 + m] for m in range(3)]

    d = v1[0] * v2[0] + v1[1] * v2[1] + v1[2] * v2[2]
    se = _expand(s1, b)
    c1 = _fold(wt[0:1024] * se, b)
    c2 = _fold(wt[1024:2048] * se, b)
    c3 = [_fold(wt[2048:3072] * _expand(v1[m], b), b) for m in range(3)]
    c4 = _fold(wt[3072:4096] * _expand(d, b), b)

    out0 = A0 * (c1 * s2 + ISQ3 * c4)
    o1 = [A1 * ISQ3 * (c2 * v2[m] + c3[m] * s2) for m in range(3)]
    tpt = jnp.concatenate([out0] + o1, axis=0)
    tp_ref[...] = tpt.T


def _tc_compute(x, eat, esht, w1t, w2t):
    grid = (EP // TCB,)
    return pl.pallas_call(
        _tc_compute_body,
        grid=grid,
        in_specs=[
            pl.BlockSpec((TCB, 128), lambda i: (i, 0)),
            pl.BlockSpec((17, TCB), lambda i: (0, i)),
            pl.BlockSpec((4, TCB), lambda i: (0, i)),
            pl.BlockSpec((64, 17), lambda i: (0, 0)),
            pl.BlockSpec((4096, 65), lambda i: (0, 0)),
        ],
        out_specs=pl.BlockSpec((TCB, 128), lambda i: (i, 0)),
        out_shape=jax.ShapeDtypeStruct((EP, 128), jnp.float32),
        compiler_params=pltpu.CompilerParams(
            dimension_semantics=("arbitrary",)),
    )(x, eat, esht, w1t, w2t)


FNB = 1000


def _tc_final_body(s0_ref, s1_ref, c0_ref, c1_ref, nap_ref, m_ref, out_ref):
    cnt = c0_ref[:, 0:1] + c1_ref[:, 0:1]
    denom = jnp.maximum(cnt, 1.0)
    res = (s0_ref[...] + s1_ref[...]) / denom + nap_ref[...]
    out_ref[...] = jnp.dot(res, m_ref[...], preferred_element_type=jnp.float32)


def _tc_finalize(s0, s1, c0, c1, nap, mperm):
    grid = (N_NODES // FNB,)
    return pl.pallas_call(
        _tc_final_body,
        grid=grid,
        in_specs=[
            pl.BlockSpec((FNB, 128), lambda i: (i, 0)),
            pl.BlockSpec((FNB, 128), lambda i: (i, 0)),
            pl.BlockSpec((FNB, 128), lambda i: (i, 0)),
            pl.BlockSpec((FNB, 128), lambda i: (i, 0)),
            pl.BlockSpec((FNB, 128), lambda i: (i, 0)),
            pl.BlockSpec((128, 128), lambda i: (0, 0)),
        ],
        out_specs=pl.BlockSpec((FNB, 128), lambda i: (i, 0)),
        out_shape=jax.ShapeDtypeStruct((N_NODES, 128), jnp.float32),
        compiler_params=pltpu.CompilerParams(
            dimension_semantics=("arbitrary",)),
    )(s0, s1, c0, c1, nap, mperm)


def kernel(node_attr, edge_index, edge_attr, edge_sh, W1, b1, W2, b2):
    f32 = jnp.float32
    nap = node_attr[:, _PERM]
    edge_dst = edge_index[0]
    edge_src = edge_index[1]
    pad = EP - N_EDGES

    dst_pad = jnp.concatenate(
        [edge_dst, jnp.zeros((pad,), jnp.int32)]).reshape(EP // CHUNK, CHUNK)
    src_pad = jnp.concatenate(
        [edge_src, jnp.full((pad,), N_NODES, jnp.int32)]
    ).reshape(EP // CHUNK, CHUNK)

    eat = jnp.pad(
        jnp.concatenate([edge_attr.T, jnp.ones((1, N_EDGES), f32)], axis=0),
        ((0, 0), (0, pad)))
    esht = jnp.pad(edge_sh.T, ((0, 0), (0, pad)))
    w1t = jnp.concatenate([W1, b1[None, :]], axis=0).T
    w2t = jnp.concatenate([W2, b2[None, :]], axis=0).T.astype(jnp.bfloat16)

    x = _sc_gather()(nap, dst_pad)
    tp = _tc_compute(x, eat, esht, w1t, w2t)

    zs = jnp.zeros((SN, 128), f32)
    ones = jnp.ones((CHUNK, 128), f32)
    sums, cnts = _sc_scatter()(tp, src_pad, zs, ones)

    mperm = jnp.zeros((128, 128), f32).at[np.arange(128), _PERM].set(1.0)

    return _tc_finalize(sums[0, :N_NODES], sums[1, :N_NODES],
                        cnts[0, :N_NODES], cnts[1, :N_NODES], nap, mperm)


# speedup vs baseline: 4.7630x; 1.0387x over previous
import functools

import jax
import jax.numpy as jnp
import numpy as np
from jax import lax
from jax.experimental import pallas as pl
from jax.experimental.pallas import tpu as pltpu
from jax.experimental.pallas import tpu_sc as plsc

N_NODES = 10000
N_EDGES = 160000
MUL = 32

NW = 32
NBUF = 4
CHUNK = 128
CHUNKS = 40
EP = NW * CHUNKS * CHUNK
SN = 10112
STRIPE = SN // 16

ISQ3 = float(1.0 / np.sqrt(3.0))
A0 = float(np.sqrt(1.0 / 64.0))
A1 = float(np.sqrt(3.0 / 64.0))

@functools.cache
def _mesh():
    return plsc.VectorSubcoreMesh(core_axis_name="c", subcore_axis_name="s",
                                  num_cores=2, num_subcores=16)

_PERM = np.concatenate([np.arange(32)] + [32 + 3 * np.arange(32) + m for m in range(3)])


def _gather_body(nap_hbm, idx_hbm, out_hbm, idx_vm, rows_v, sem0, sem1):
    c = lax.axis_index("c")
    s = lax.axis_index("s")
    wid = s * 2 + c
    sems = (sem0, sem1)

    pltpu.sync_copy(idx_hbm.at[wid], idx_vm)

    def body(g, carry):
        j0 = NBUF * g
        gd = [pltpu.async_copy(nap_hbm.at[idx_vm.at[j0 + b]], rows_v.at[b],
                               sems[0]) for b in range(NBUF)]
        for d in gd:
            d.wait()
        wd = [pltpu.async_copy(
            rows_v.at[b],
            out_hbm.at[pl.ds((wid * CHUNKS + j0 + b) * CHUNK, CHUNK)],
            sems[1]) for b in range(NBUF)]
        for d in wd:
            d.wait()
        return carry

    lax.fori_loop(0, CHUNKS // NBUF, body, 0)


@functools.cache
def _sc_gather():
    return pl.kernel(
        _gather_body,
        out_type=jax.ShapeDtypeStruct((EP, 128), jnp.float32),
        mesh=_mesh(),
        scratch_types=[
            pltpu.VMEM((CHUNKS, CHUNK), jnp.int32),
            pltpu.VMEM((NBUF, CHUNK, 128), jnp.float32),
            pltpu.SemaphoreType.DMA,
            pltpu.SemaphoreType.DMA,
        ],
    )


def _scatter_body(tp_hbm, src_hbm, zs_hbm, ones_hbm, sums_out, cnts_out,
                  i0, i1, i2, i3, r0, r1, r2, r3, ones_v, acc_sh,
                  sem_i, sem_r, sem_s):
    c = lax.axis_index("c")
    s = lax.axis_index("s")
    wid = s * 2 + c
    idxs = (i0, i1, i2, i3)
    rows = (r0, r1, r2, r3)
    pltpu.sync_copy(ones_hbm, ones_v)

    def phase(out_ref, scatter_tp):
        pltpu.sync_copy(zs_hbm.at[pl.ds(s * STRIPE, STRIPE)],
                        acc_sh.at[pl.ds(s * STRIPE, STRIPE)])
        plsc.subcore_barrier()

        def body(j, carry):
            r = wid * CHUNKS + j
            pltpu.sync_copy(src_hbm.at[r], i0)
            if scatter_tp:
                pltpu.sync_copy(tp_hbm.at[pl.ds(r * CHUNK, CHUNK)], r0)
                pltpu.sync_copy(r0, acc_sh.at[i0], add=True)
            else:
                pltpu.sync_copy(ones_v, acc_sh.at[i0], add=True)
            return carry

        lax.fori_loop(0, CHUNKS, body, 0)
        plsc.subcore_barrier()
        pltpu.sync_copy(acc_sh.at[pl.ds(s * STRIPE, STRIPE)],
                        out_ref.at[c, pl.ds(s * STRIPE, STRIPE)])
        plsc.subcore_barrier()

    phase(sums_out, True)
    phase(cnts_out, False)


@functools.cache
def _sc_scatter():
    return pl.kernel(
        _scatter_body,
        out_type=(
            jax.ShapeDtypeStruct((2, SN, 128), jnp.float32),
            jax.ShapeDtypeStruct((2, SN, 128), jnp.float32),
        ),
        mesh=_mesh(),
        scratch_types=[
            pltpu.VMEM((CHUNK,), jnp.int32),
            pltpu.VMEM((CHUNK,), jnp.int32),
            pltpu.VMEM((CHUNK,), jnp.int32),
            pltpu.VMEM((CHUNK,), jnp.int32),
            pltpu.VMEM((CHUNK, 128), jnp.float32),
            pltpu.VMEM((CHUNK, 128), jnp.float32),
            pltpu.VMEM((CHUNK, 128), jnp.float32),
            pltpu.VMEM((CHUNK, 128), jnp.float32),
            pltpu.VMEM((CHUNK, 128), jnp.float32),
            pltpu.VMEM_SHARED((SN, 128), jnp.float32),
            pltpu.SemaphoreType.DMA,
            pltpu.SemaphoreType.DMA,
            pltpu.SemaphoreType.DMA,
        ],
    )


TCB = 256


def _expand(a, b):
    return jnp.broadcast_to(a[:, None, :], (32, 32, b)).reshape(32 * 32, b)


def _fold(p, b):
    for h in (512, 256, 128, 64, 32):
        p = p[:h] + p[h:2 * h]
    return p


def _tc_compute_body(x_ref, ea_ref, esh_ref, w1t_ref, w2t_ref, tp_ref):
    b = TCB
    xt = x_ref[...].T
    ht = jnp.maximum(jnp.dot(w1t_ref[...], ea_ref[...],
                             preferred_element_type=jnp.float32), 0.0)
    hta = jnp.concatenate([ht, jnp.ones((1, b), jnp.float32)], axis=0)
    wt = jnp.dot(w2t_ref[...], hta.astype(jnp.bfloat16),
                 preferred_element_type=jnp.float32)

    s1 = xt[0:32]
    v1 = [xt[32 + 32 * m:64 + 32 * m] for m in range(3)]
    s2 = esh_ref[0:1]
    v2 = [esh_ref[1 + m:2 + m] for m in range(3)]

    d = v1[0] * v2[0] + v1[1] * v2[1] + v1[2] * v2[2]
    se = _expand(s1, b)
    c1 = _fold(wt[0:1024] * se, b)
    c2 = _fold(wt[1024:2048] * se, b)
    c3 = [_fold(wt[2048:3072] * _expand(v1[m], b), b) for m in range(3)]
    c4 = _fold(wt[3072:4096] * _expand(d, b), b)

    out0 = A0 * (c1 * s2 + ISQ3 * c4)
    o1 = [A1 * ISQ3 * (c2 * v2[m] + c3[m] * s2) for m in range(3)]
    tpt = jnp.concatenate([out0] + o1, axis=0)
    tp_ref[...] = tpt.T


def _tc_compute(x, eat, esht, w1t, w2t):
    grid = (EP // TCB,)
    return pl.pallas_call(
        _tc_compute_body,
        grid=grid,
        in_specs=[
            pl.BlockSpec((TCB, 128), lambda i: (i, 0)),
            pl.BlockSpec((17, TCB), lambda i: (0, i)),
            pl.BlockSpec((4, TCB), lambda i: (0, i)),
            pl.BlockSpec((64, 17), lambda i: (0, 0)),
            pl.BlockSpec((4096, 65), lambda i: (0, 0)),
        ],
        out_specs=pl.BlockSpec((TCB, 128), lambda i: (i, 0)),
        out_shape=jax.ShapeDtypeStruct((EP, 128), jnp.float32),
        compiler_params=pltpu.CompilerParams(
            dimension_semantics=("arbitrary",)),
    )(x, eat, esht, w1t, w2t)


FNB = 1000


def _tc_final_body(s0_ref, s1_ref, c0_ref, c1_ref, nap_ref, m_ref, out_ref):
    cnt = c0_ref[:, 0:1] + c1_ref[:, 0:1]
    denom = jnp.maximum(cnt, 1.0)
    res = (s0_ref[...] + s1_ref[...]) / denom + nap_ref[...]
    out_ref[...] = jnp.dot(res, m_ref[...], preferred_element_type=jnp.float32)


def _tc_finalize(s0, s1, c0, c1, nap, mperm):
    grid = (N_NODES // FNB,)
    return pl.pallas_call(
        _tc_final_body,
        grid=grid,
        in_specs=[
            pl.BlockSpec((FNB, 128), lambda i: (i, 0)),
            pl.BlockSpec((FNB, 128), lambda i: (i, 0)),
            pl.BlockSpec((FNB, 128), lambda i: (i, 0)),
            pl.BlockSpec((FNB, 128), lambda i: (i, 0)),
            pl.BlockSpec((FNB, 128), lambda i: (i, 0)),
            pl.BlockSpec((128, 128), lambda i: (0, 0)),
        ],
        out_specs=pl.BlockSpec((FNB, 128), lambda i: (i, 0)),
        out_shape=jax.ShapeDtypeStruct((N_NODES, 128), jnp.float32),
        compiler_params=pltpu.CompilerParams(
            dimension_semantics=("arbitrary",)),
    )(s0, s1, c0, c1, nap, mperm)


def kernel(node_attr, edge_index, edge_attr, edge_sh, W1, b1, W2, b2):
    f32 = jnp.float32
    nap = node_attr[:, _PERM]
    edge_dst = edge_index[0]
    edge_src = edge_index[1]
    pad = EP - N_EDGES

    dst_pad = jnp.concatenate(
        [edge_dst, jnp.zeros((pad,), jnp.int32)]).reshape(NW, CHUNKS, CHUNK)
    src_pad = jnp.concatenate(
        [edge_src, jnp.full((pad,), N_NODES, jnp.int32)]
    ).reshape(EP // CHUNK, CHUNK)

    eat = jnp.pad(
        jnp.concatenate([edge_attr.T, jnp.ones((1, N_EDGES), f32)], axis=0),
        ((0, 0), (0, pad)))
    esht = jnp.pad(edge_sh.T, ((0, 0), (0, pad)))
    w1t = jnp.concatenate([W1, b1[None, :]], axis=0).T
    w2t = jnp.concatenate([W2, b2[None, :]], axis=0).T.astype(jnp.bfloat16)

    x = _sc_gather()(nap, dst_pad)
    tp = _tc_compute(x, eat, esht, w1t, w2t)

    zs = jnp.zeros((SN, 128), f32)
    ones = jnp.ones((CHUNK, 128), f32)
    sums, cnts = _sc_scatter()(tp, src_pad, zs, ones)

    mperm = jnp.zeros((128, 128), f32).at[np.arange(128), _PERM].set(1.0)

    return _tc_finalize(sums[0, :N_NODES], sums[1, :N_NODES],
                        cnts[0, :N_NODES], cnts[1, :N_NODES], nap, mperm)


# speedup vs baseline: 5.2380x; 1.0997x over previous
import functools

import jax
import jax.numpy as jnp
import numpy as np
from jax import lax
from jax.experimental import pallas as pl
from jax.experimental.pallas import tpu as pltpu
from jax.experimental.pallas import tpu_sc as plsc

N_NODES = 10000
N_EDGES = 160000
MUL = 32

NW = 32
NBUF = 4
CHUNK = 128
CHUNKS = 40
EP = NW * CHUNKS * CHUNK
SN = 10112
STRIPE = SN // 16

ISQ3 = float(1.0 / np.sqrt(3.0))
A0 = float(np.sqrt(1.0 / 64.0))
A1 = float(np.sqrt(3.0 / 64.0))

@functools.cache
def _mesh():
    return plsc.VectorSubcoreMesh(core_axis_name="c", subcore_axis_name="s",
                                  num_cores=2, num_subcores=16)

_PERM = np.concatenate([np.arange(32)] + [32 + 3 * np.arange(32) + m for m in range(3)])


def _gather_body(nap_hbm, idx_hbm, out_hbm, idx_vm, rows_v, sem0, sem1):
    c = lax.axis_index("c")
    s = lax.axis_index("s")
    wid = s * 2 + c
    sems = (sem0, sem1)

    pltpu.sync_copy(idx_hbm.at[wid], idx_vm)

    def body(g, carry):
        j0 = NBUF * g
        gd = [pltpu.async_copy(nap_hbm.at[idx_vm.at[j0 + b]], rows_v.at[b],
                               sems[0]) for b in range(NBUF)]
        for d in gd:
            d.wait()
        wd = [pltpu.async_copy(
            rows_v.at[b],
            out_hbm.at[pl.ds((wid * CHUNKS + j0 + b) * CHUNK, CHUNK)],
            sems[1]) for b in range(NBUF)]
        for d in wd:
            d.wait()
        return carry

    lax.fori_loop(0, CHUNKS // NBUF, body, 0)


@functools.cache
def _sc_gather():
    return pl.kernel(
        _gather_body,
        out_type=jax.ShapeDtypeStruct((EP, 128), jnp.float32),
        mesh=_mesh(),
        scratch_types=[
            pltpu.VMEM((CHUNKS, CHUNK), jnp.int32),
            pltpu.VMEM((NBUF, CHUNK, 128), jnp.float32),
            pltpu.SemaphoreType.DMA,
            pltpu.SemaphoreType.DMA,
        ],
    )


def _scatter_sums_body(tp_hbm, src_hbm, zs_hbm, sums_out,
                       i0, i1, r0, r1, acc_sh, sem_i, sem_r):
    c = lax.axis_index("c")
    s = lax.axis_index("s")
    wid = s * 2 + c
    idxs = (i0, i1)
    rows = (r0, r1)

    pltpu.sync_copy(zs_hbm.at[pl.ds(s * STRIPE, STRIPE)],
                    acc_sh.at[pl.ds(s * STRIPE, STRIPE)])
    plsc.subcore_barrier()

    def body(g, carry):
        j0 = 2 * g
        ld = []
        for b in range(2):
            r = wid * CHUNKS + j0 + b
            ld.append(pltpu.async_copy(src_hbm.at[r], idxs[b], sem_i))
            ld.append(pltpu.async_copy(tp_hbm.at[pl.ds(r * CHUNK, CHUNK)],
                                       rows[b], sem_r))
        for b in range(2):
            ld[2 * b].wait()
            ld[2 * b + 1].wait()
            pltpu.sync_copy(rows[b], acc_sh.at[idxs[b]], add=True)
        return carry

    lax.fori_loop(0, CHUNKS // 2, body, 0)
    plsc.subcore_barrier()
    pltpu.sync_copy(acc_sh.at[pl.ds(s * STRIPE, STRIPE)],
                    sums_out.at[c, pl.ds(s * STRIPE, STRIPE)])


def _scatter_cnts_body(src_hbm, zs_hbm, ones_hbm, cnts_out,
                       i0, ones_v, acc_sh):
    c = lax.axis_index("c")
    s = lax.axis_index("s")
    wid = s * 2 + c
    pltpu.sync_copy(ones_hbm, ones_v)
    pltpu.sync_copy(zs_hbm.at[pl.ds(s * STRIPE, STRIPE)],
                    acc_sh.at[pl.ds(s * STRIPE, STRIPE)])
    plsc.subcore_barrier()

    def body(j, carry):
        r = wid * CHUNKS + j
        pltpu.sync_copy(src_hbm.at[r], i0)
        pltpu.sync_copy(ones_v, acc_sh.at[i0], add=True)
        return carry

    lax.fori_loop(0, CHUNKS, body, 0)
    plsc.subcore_barrier()
    pltpu.sync_copy(acc_sh.at[pl.ds(s * STRIPE, STRIPE)],
                    cnts_out.at[c, pl.ds(s * STRIPE, STRIPE)])


@functools.cache
def _sc_scatter_sums():
    return pl.kernel(
        _scatter_sums_body,
        out_type=jax.ShapeDtypeStruct((2, SN, 128), jnp.float32),
        mesh=_mesh(),
        scratch_types=[
            pltpu.VMEM((CHUNK,), jnp.int32),
            pltpu.VMEM((CHUNK,), jnp.int32),
            pltpu.VMEM((CHUNK, 128), jnp.float32),
            pltpu.VMEM((CHUNK, 128), jnp.float32),
            pltpu.VMEM_SHARED((SN, 128), jnp.float32),
            pltpu.SemaphoreType.DMA,
            pltpu.SemaphoreType.DMA,
        ],
    )


@functools.cache
def _sc_scatter_cnts():
    return pl.kernel(
        _scatter_cnts_body,
        out_type=jax.ShapeDtypeStruct((2, SN, 128), jnp.float32),
        mesh=_mesh(),
        scratch_types=[
            pltpu.VMEM((CHUNK,), jnp.int32),
            pltpu.VMEM((CHUNK, 128), jnp.float32),
            pltpu.VMEM_SHARED((SN, 128), jnp.float32),
        ],
    )


TCB = 256


def _expand(a, b):
    return jnp.broadcast_to(a[:, None, :], (32, 32, b)).reshape(32 * 32, b)


def _fold(p, b):
    for h in (512, 256, 128, 64, 32):
        p = p[:h] + p[h:2 * h]
    return p


def _tc_compute_body(x_ref, ea_ref, esh_ref, w1t_ref, w2t_ref, tp_ref):
    b = TCB
    xt = x_ref[...].T
    ht = jnp.maximum(jnp.dot(w1t_ref[...], ea_ref[...],
                             preferred_element_type=jnp.float32), 0.0)
    hta = jnp.concatenate([ht, jnp.ones((1, b), jnp.float32)], axis=0)
    wt = jnp.dot(w2t_ref[...], hta.astype(jnp.bfloat16),
                 preferred_element_type=jnp.float32)

    s1 = xt[0:32]
    v1 = [xt[32 + 32 * m:64 + 32 * m] for m in range(3)]
    s2 = esh_ref[0:1]
    v2 = [esh_ref[1 + m:2 + m] for m in range(3)]

    d = v1[0] * v2[0] + v1[1] * v2[1] + v1[2] * v2[2]
    se = _expand(s1, b)
    c1 = _fold(wt[0:1024] * se, b)
    c2 = _fold(wt[1024:2048] * se, b)
    c3 = [_fold(wt[2048:3072] * _expand(v1[m], b), b) for m in range(3)]
    c4 = _fold(wt[3072:4096] * _expand(d, b), b)

    out0 = A0 * (c1 * s2 + ISQ3 * c4)
    o1 = [A1 * ISQ3 * (c2 * v2[m] + c3[m] * s2) for m in range(3)]
    tpt = jnp.concatenate([out0] + o1, axis=0)
    tp_ref[...] = tpt.T


def _tc_compute(x, eat, esht, w1t, w2t):
    grid = (EP // TCB,)
    return pl.pallas_call(
        _tc_compute_body,
        grid=grid,
        in_specs=[
            pl.BlockSpec((TCB, 128), lambda i: (i, 0)),
            pl.BlockSpec((17, TCB), lambda i: (0, i)),
            pl.BlockSpec((4, TCB), lambda i: (0, i)),
            pl.BlockSpec((64, 17), lambda i: (0, 0)),
            pl.BlockSpec((4096, 65), lambda i: (0, 0)),
        ],
        out_specs=pl.BlockSpec((TCB, 128), lambda i: (i, 0)),
        out_shape=jax.ShapeDtypeStruct((EP, 128), jnp.float32),
        compiler_params=pltpu.CompilerParams(
            dimension_semantics=("arbitrary",)),
    )(x, eat, esht, w1t, w2t)


FNB = 1000


def _tc_final_body(s0_ref, s1_ref, c0_ref, c1_ref, nap_ref, m_ref, out_ref):
    cnt = c0_ref[:, 0:1] + c1_ref[:, 0:1]
    denom = jnp.maximum(cnt, 1.0)
    res = (s0_ref[...] + s1_ref[...]) / denom + nap_ref[...]
    out_ref[...] = jnp.dot(res, m_ref[...], preferred_element_type=jnp.float32)


def _tc_finalize(s0, s1, c0, c1, nap, mperm):
    grid = (N_NODES // FNB,)
    return pl.pallas_call(
        _tc_final_body,
        grid=grid,
        in_specs=[
            pl.BlockSpec((FNB, 128), lambda i: (i, 0)),
            pl.BlockSpec((FNB, 128), lambda i: (i, 0)),
            pl.BlockSpec((FNB, 128), lambda i: (i, 0)),
            pl.BlockSpec((FNB, 128), lambda i: (i, 0)),
            pl.BlockSpec((FNB, 128), lambda i: (i, 0)),
            pl.BlockSpec((128, 128), lambda i: (0, 0)),
        ],
        out_specs=pl.BlockSpec((FNB, 128), lambda i: (i, 0)),
        out_shape=jax.ShapeDtypeStruct((N_NODES, 128), jnp.float32),
        compiler_params=pltpu.CompilerParams(
            dimension_semantics=("arbitrary",)),
    )(s0, s1, c0, c1, nap, mperm)


def kernel(node_attr, edge_index, edge_attr, edge_sh, W1, b1, W2, b2):
    f32 = jnp.float32
    nap = node_attr[:, _PERM]
    edge_dst = edge_index[0]
    edge_src = edge_index[1]
    pad = EP - N_EDGES

    dst_pad = jnp.concatenate(
        [edge_dst, jnp.zeros((pad,), jnp.int32)]).reshape(NW, CHUNKS, CHUNK)
    src_pad = jnp.concatenate(
        [edge_src, jnp.full((pad,), N_NODES, jnp.int32)]
    ).reshape(EP // CHUNK, CHUNK)

    eat = jnp.pad(
        jnp.concatenate([edge_attr.T, jnp.ones((1, N_EDGES), f32)], axis=0),
        ((0, 0), (0, pad)))
    esht = jnp.pad(edge_sh.T, ((0, 0), (0, pad)))
    w1t = jnp.concatenate([W1, b1[None, :]], axis=0).T
    w2t = jnp.concatenate([W2, b2[None, :]], axis=0).T.astype(jnp.bfloat16)

    zs = jnp.zeros((SN, 128), f32)
    ones = jnp.ones((CHUNK, 128), f32)
    cnts = _sc_scatter_cnts()(src_pad, zs, ones)
    x = _sc_gather()(nap, dst_pad)
    tp = _tc_compute(x, eat, esht, w1t, w2t)
    sums = _sc_scatter_sums()(tp, src_pad, zs)

    mperm = jnp.zeros((128, 128), f32).at[np.arange(128), _PERM].set(1.0)

    return _tc_finalize(sums[0, :N_NODES], sums[1, :N_NODES],
                        cnts[0, :N_NODES], cnts[1, :N_NODES], nap, mperm)


# speedup vs baseline: 5.5796x; 1.0652x over previous
import functools

import jax
import jax.numpy as jnp
import numpy as np
from jax import lax
from jax.experimental import pallas as pl
from jax.experimental.pallas import tpu as pltpu
from jax.experimental.pallas import tpu_sc as plsc

N_NODES = 10000
N_EDGES = 160000
MUL = 32

NW = 32
NBUF = 4
CHUNK = 128
CHUNKS = 40
EP = NW * CHUNKS * CHUNK
SN = 10112
STRIPE = SN // 16

ISQ3 = float(1.0 / np.sqrt(3.0))
A0 = float(np.sqrt(1.0 / 64.0))
A1 = float(np.sqrt(3.0 / 64.0))

@functools.cache
def _mesh():
    return plsc.VectorSubcoreMesh(core_axis_name="c", subcore_axis_name="s",
                                  num_cores=2, num_subcores=16)

_PERM = np.concatenate([np.arange(32)] + [32 + 3 * np.arange(32) + m for m in range(3)])


def _gather_body(nap_hbm, idx_hbm, out_hbm, idx_vm, rows_v, sem0, sem1):
    c = lax.axis_index("c")
    s = lax.axis_index("s")
    wid = s * 2 + c
    sems = (sem0, sem1)

    pltpu.sync_copy(idx_hbm.at[wid], idx_vm)

    def body(g, carry):
        j0 = NBUF * g
        gd = [pltpu.async_copy(nap_hbm.at[idx_vm.at[j0 + b]], rows_v.at[b],
                               sems[0]) for b in range(NBUF)]
        for d in gd:
            d.wait()
        wd = [pltpu.async_copy(
            rows_v.at[b],
            out_hbm.at[pl.ds((wid * CHUNKS + j0 + b) * CHUNK, CHUNK)],
            sems[1]) for b in range(NBUF)]
        for d in wd:
            d.wait()
        return carry

    lax.fori_loop(0, CHUNKS // NBUF, body, 0)


@functools.cache
def _sc_gather():
    return pl.kernel(
        _gather_body,
        out_type=jax.ShapeDtypeStruct((EP, 128), jnp.float32),
        mesh=_mesh(),
        scratch_types=[
            pltpu.VMEM((CHUNKS, CHUNK), jnp.int32),
            pltpu.VMEM((NBUF, CHUNK, 128), jnp.float32),
            pltpu.SemaphoreType.DMA,
            pltpu.SemaphoreType.DMA,
        ],
    )


def _scatter_sums_body(tp_hbm, src_hbm, zs_hbm, sums_out,
                       i0, i1, r0, r1, acc_sh, sem_i, sem_r):
    c = lax.axis_index("c")
    s = lax.axis_index("s")
    wid = s * 2 + c
    idxs = (i0, i1)
    rows = (r0, r1)

    pltpu.sync_copy(zs_hbm.at[pl.ds(s * STRIPE, STRIPE)],
                    acc_sh.at[pl.ds(s * STRIPE, STRIPE)])
    plsc.subcore_barrier()

    def body(g, carry):
        j0 = 2 * g
        ld = []
        for b in range(2):
            r = wid * CHUNKS + j0 + b
            ld.append(pltpu.async_copy(src_hbm.at[r], idxs[b], sem_i))
            ld.append(pltpu.async_copy(tp_hbm.at[pl.ds(r * CHUNK, CHUNK)],
                                       rows[b], sem_r))
        for b in range(2):
            ld[2 * b].wait()
            ld[2 * b + 1].wait()
            pltpu.sync_copy(rows[b], acc_sh.at[idxs[b]], add=True)
        return carry

    lax.fori_loop(0, CHUNKS // 2, body, 0)
    plsc.subcore_barrier()
    pltpu.sync_copy(acc_sh.at[pl.ds(s * STRIPE, STRIPE)],
                    sums_out.at[c, pl.ds(s * STRIPE, STRIPE)])


def _scatter_cnts_body(src_hbm, zs_hbm, ones_hbm, cnts_out,
                       i0, ones_v, acc_sh):
    c = lax.axis_index("c")
    s = lax.axis_index("s")
    wid = s * 2 + c
    pltpu.sync_copy(ones_hbm, ones_v)
    pltpu.sync_copy(zs_hbm.at[pl.ds(s * STRIPE, STRIPE)],
                    acc_sh.at[pl.ds(s * STRIPE, STRIPE)])
    plsc.subcore_barrier()

    def body(j, carry):
        r = wid * CHUNKS + j
        pltpu.sync_copy(src_hbm.at[r], i0)
        pltpu.sync_copy(ones_v, acc_sh.at[i0], add=True)
        return carry

    lax.fori_loop(0, CHUNKS, body, 0)
    plsc.subcore_barrier()
    pltpu.sync_copy(acc_sh.at[pl.ds(s * STRIPE, STRIPE)],
                    cnts_out.at[c, pl.ds(s * STRIPE, STRIPE)])


@functools.cache
def _sc_scatter_sums():
    return pl.kernel(
        _scatter_sums_body,
        out_type=jax.ShapeDtypeStruct((2, SN, 128), jnp.float32),
        mesh=_mesh(),
        scratch_types=[
            pltpu.VMEM((CHUNK,), jnp.int32),
            pltpu.VMEM((CHUNK,), jnp.int32),
            pltpu.VMEM((CHUNK, 128), jnp.float32),
            pltpu.VMEM((CHUNK, 128), jnp.float32),
            pltpu.VMEM_SHARED((SN, 128), jnp.float32),
            pltpu.SemaphoreType.DMA,
            pltpu.SemaphoreType.DMA,
        ],
    )


@functools.cache
def _sc_scatter_cnts():
    return pl.kernel(
        _scatter_cnts_body,
        out_type=jax.ShapeDtypeStruct((2, SN, 128), jnp.float32),
        mesh=_mesh(),
        scratch_types=[
            pltpu.VMEM((CHUNK,), jnp.int32),
            pltpu.VMEM((CHUNK, 128), jnp.float32),
            pltpu.VMEM_SHARED((SN, 128), jnp.float32),
        ],
    )


TCB = 256


def _expand(a, b):
    return jnp.broadcast_to(a[:, None, :], (32, 32, b)).reshape(32 * 32, b)


def _fold(p, b):
    for h in (512, 256, 128, 64, 32):
        p = p[:h] + p[h:2 * h]
    return p


def _tc_compute_body(x_ref, ea_ref, esh_ref, w1t_ref, w2t_ref, tp_ref):
    b = TCB
    xt = x_ref[...].T
    ht = jnp.maximum(jnp.dot(w1t_ref[...], ea_ref[...],
                             preferred_element_type=jnp.float32), 0.0)
    hta = jnp.concatenate([ht, jnp.ones((1, b), jnp.float32)], axis=0)
    wt = jnp.dot(w2t_ref[...], hta.astype(jnp.bfloat16),
                 preferred_element_type=jnp.float32)

    s1 = xt[0:32]
    v1 = [xt[32 + 32 * m:64 + 32 * m] for m in range(3)]
    s2 = esh_ref[0:1]
    v2 = [esh_ref[1 + m:2 + m] for m in range(3)]

    d = v1[0] * v2[0] + v1[1] * v2[1] + v1[2] * v2[2]

    def contract(base, coef):
        acc = wt[base:base + 32] * coef[0:1]
        for u in range(1, 32):
            acc = acc + wt[base + 32 * u:base + 32 * (u + 1)] * coef[u:u + 1]
        return acc

    c1 = contract(0, s1)
    c2 = contract(1024, s1)
    c3 = [contract(2048, v1[m]) for m in range(3)]
    c4 = contract(3072, d)

    out0 = A0 * (c1 * s2 + ISQ3 * c4)
    o1 = [A1 * ISQ3 * (c2 * v2[m] + c3[m] * s2) for m in range(3)]
    tpt = jnp.concatenate([out0] + o1, axis=0)
    tp_ref[...] = tpt.T


def _tc_compute(x, eat, esht, w1t, w2t):
    grid = (EP // TCB,)
    return pl.pallas_call(
        _tc_compute_body,
        grid=grid,
        in_specs=[
            pl.BlockSpec((TCB, 128), lambda i: (i, 0)),
            pl.BlockSpec((17, TCB), lambda i: (0, i)),
            pl.BlockSpec((4, TCB), lambda i: (0, i)),
            pl.BlockSpec((64, 17), lambda i: (0, 0)),
            pl.BlockSpec((4096, 65), lambda i: (0, 0)),
        ],
        out_specs=pl.BlockSpec((TCB, 128), lambda i: (i, 0)),
        out_shape=jax.ShapeDtypeStruct((EP, 128), jnp.float32),
        compiler_params=pltpu.CompilerParams(
            dimension_semantics=("arbitrary",)),
    )(x, eat, esht, w1t, w2t)


FNB = 1000


def _tc_final_body(s0_ref, s1_ref, c0_ref, c1_ref, nap_ref, m_ref, out_ref):
    cnt = c0_ref[:, 0:1] + c1_ref[:, 0:1]
    denom = jnp.maximum(cnt, 1.0)
    res = (s0_ref[...] + s1_ref[...]) / denom + nap_ref[...]
    out_ref[...] = jnp.dot(res, m_ref[...], preferred_element_type=jnp.float32)


def _tc_finalize(s0, s1, c0, c1, nap, mperm):
    grid = (N_NODES // FNB,)
    return pl.pallas_call(
        _tc_final_body,
        grid=grid,
        in_specs=[
            pl.BlockSpec((FNB, 128), lambda i: (i, 0)),
            pl.BlockSpec((FNB, 128), lambda i: (i, 0)),
            pl.BlockSpec((FNB, 128), lambda i: (i, 0)),
            pl.BlockSpec((FNB, 128), lambda i: (i, 0)),
            pl.BlockSpec((FNB, 128), lambda i: (i, 0)),
            pl.BlockSpec((128, 128), lambda i: (0, 0)),
        ],
        out_specs=pl.BlockSpec((FNB, 128), lambda i: (i, 0)),
        out_shape=jax.ShapeDtypeStruct((N_NODES, 128), jnp.float32),
        compiler_params=pltpu.CompilerParams(
            dimension_semantics=("arbitrary",)),
    )(s0, s1, c0, c1, nap, mperm)


def kernel(node_attr, edge_index, edge_attr, edge_sh, W1, b1, W2, b2):
    f32 = jnp.float32
    nap = node_attr[:, _PERM]
    edge_dst = edge_index[0]
    edge_src = edge_index[1]
    pad = EP - N_EDGES

    dst_pad = jnp.concatenate(
        [edge_dst, jnp.zeros((pad,), jnp.int32)]).reshape(NW, CHUNKS, CHUNK)
    src_pad = jnp.concatenate(
        [edge_src, jnp.full((pad,), N_NODES, jnp.int32)]
    ).reshape(EP // CHUNK, CHUNK)

    eat = jnp.pad(
        jnp.concatenate([edge_attr.T, jnp.ones((1, N_EDGES), f32)], axis=0),
        ((0, 0), (0, pad)))
    esht = jnp.pad(edge_sh.T, ((0, 0), (0, pad)))
    w1t = jnp.concatenate([W1, b1[None, :]], axis=0).T
    w2t = jnp.concatenate([W2, b2[None, :]], axis=0).T.astype(jnp.bfloat16)

    zs = jnp.zeros((SN, 128), f32)
    ones = jnp.ones((CHUNK, 128), f32)
    cnts = _sc_scatter_cnts()(src_pad, zs, ones)
    x = _sc_gather()(nap, dst_pad)
    tp = _tc_compute(x, eat, esht, w1t, w2t)
    sums = _sc_scatter_sums()(tp, src_pad, zs)

    mperm = jnp.zeros((128, 128), f32).at[np.arange(128), _PERM].set(1.0)

    return _tc_finalize(sums[0, :N_NODES], sums[1, :N_NODES],
                        cnts[0, :N_NODES], cnts[1, :N_NODES], nap, mperm)


# speedup vs baseline: 6.0957x; 1.0925x over previous
import functools

import jax
import jax.numpy as jnp
import numpy as np
from jax import lax
from jax.experimental import pallas as pl
from jax.experimental.pallas import tpu as pltpu
from jax.experimental.pallas import tpu_sc as plsc

N_NODES = 10000
N_EDGES = 160000
MUL = 32

NW = 32
NBUF = 4
CHUNK = 128
CHUNKS = 40
EP = NW * CHUNKS * CHUNK
SN = 10112
STRIPE = SN // 16

ISQ3 = float(1.0 / np.sqrt(3.0))
A0 = float(np.sqrt(1.0 / 64.0))
A1 = float(np.sqrt(3.0 / 64.0))

@functools.cache
def _mesh():
    return plsc.VectorSubcoreMesh(core_axis_name="c", subcore_axis_name="s",
                                  num_cores=2, num_subcores=16)

_PERM = np.concatenate([np.arange(32)] + [32 + 3 * np.arange(32) + m for m in range(3)])


def _gather_body(nap_hbm, idx_hbm, out_hbm, idx_vm, rows_v, sem0, sem1):
    c = lax.axis_index("c")
    s = lax.axis_index("s")
    wid = s * 2 + c
    sems = (sem0, sem1)

    pltpu.sync_copy(idx_hbm.at[wid], idx_vm)

    def body(g, carry):
        j0 = NBUF * g
        gd = [pltpu.async_copy(nap_hbm.at[idx_vm.at[j0 + b]], rows_v.at[b],
                               sems[0]) for b in range(NBUF)]
        for d in gd:
            d.wait()
        wd = [pltpu.async_copy(
            rows_v.at[b],
            out_hbm.at[pl.ds((wid * CHUNKS + j0 + b) * CHUNK, CHUNK)],
            sems[1]) for b in range(NBUF)]
        for d in wd:
            d.wait()
        return carry

    lax.fori_loop(0, CHUNKS // NBUF, body, 0)


@functools.cache
def _sc_gather():
    return pl.kernel(
        _gather_body,
        out_type=jax.ShapeDtypeStruct((EP, 128), jnp.float32),
        mesh=_mesh(),
        scratch_types=[
            pltpu.VMEM((CHUNKS, CHUNK), jnp.int32),
            pltpu.VMEM((NBUF, CHUNK, 128), jnp.float32),
            pltpu.SemaphoreType.DMA,
            pltpu.SemaphoreType.DMA,
        ],
    )


def _scatter_sums_body(tp_hbm, src_hbm, zs_hbm, sums_out,
                       i0, i1, r0, r1, acc_sh, sem_i, sem_r):
    c = lax.axis_index("c")
    s = lax.axis_index("s")
    wid = s * 2 + c
    idxs = (i0, i1)
    rows = (r0, r1)

    pltpu.sync_copy(zs_hbm.at[pl.ds(s * STRIPE, STRIPE)],
                    acc_sh.at[pl.ds(s * STRIPE, STRIPE)])
    plsc.subcore_barrier()

    def body(g, carry):
        j0 = 2 * g
        ld = []
        for b in range(2):
            r = wid * CHUNKS + j0 + b
            ld.append(pltpu.async_copy(src_hbm.at[r], idxs[b], sem_i))
            ld.append(pltpu.async_copy(tp_hbm.at[pl.ds(r * CHUNK, CHUNK)],
                                       rows[b], sem_r))
        for b in range(2):
            ld[2 * b].wait()
            ld[2 * b + 1].wait()
            pltpu.sync_copy(rows[b], acc_sh.at[idxs[b]], add=True)
        return carry

    lax.fori_loop(0, CHUNKS // 2, body, 0)
    plsc.subcore_barrier()
    pltpu.sync_copy(acc_sh.at[pl.ds(s * STRIPE, STRIPE)],
                    sums_out.at[c, pl.ds(s * STRIPE, STRIPE)])


def _scatter_cnts_body(src_hbm, zs_hbm, ones_hbm, cnts_out,
                       i0, ones_v, acc_sh):
    c = lax.axis_index("c")
    s = lax.axis_index("s")
    wid = s * 2 + c
    pltpu.sync_copy(ones_hbm, ones_v)
    pltpu.sync_copy(zs_hbm.at[pl.ds(s * STRIPE, STRIPE)],
                    acc_sh.at[pl.ds(s * STRIPE, STRIPE)])
    plsc.subcore_barrier()

    def body(j, carry):
        r = wid * CHUNKS + j
        pltpu.sync_copy(src_hbm.at[r], i0)
        pltpu.sync_copy(ones_v, acc_sh.at[i0], add=True)
        return carry

    lax.fori_loop(0, CHUNKS, body, 0)
    plsc.subcore_barrier()
    pltpu.sync_copy(acc_sh.at[pl.ds(s * STRIPE, STRIPE)],
                    cnts_out.at[c, pl.ds(s * STRIPE, STRIPE)])


@functools.cache
def _sc_scatter_sums():
    return pl.kernel(
        _scatter_sums_body,
        out_type=jax.ShapeDtypeStruct((2, SN, 128), jnp.float32),
        mesh=_mesh(),
        scratch_types=[
            pltpu.VMEM((CHUNK,), jnp.int32),
            pltpu.VMEM((CHUNK,), jnp.int32),
            pltpu.VMEM((CHUNK, 128), jnp.float32),
            pltpu.VMEM((CHUNK, 128), jnp.float32),
            pltpu.VMEM_SHARED((SN, 128), jnp.float32),
            pltpu.SemaphoreType.DMA,
            pltpu.SemaphoreType.DMA,
        ],
    )


@functools.cache
def _sc_scatter_cnts():
    return pl.kernel(
        _scatter_cnts_body,
        out_type=jax.ShapeDtypeStruct((2, SN, 128), jnp.float32),
        mesh=_mesh(),
        scratch_types=[
            pltpu.VMEM((CHUNK,), jnp.int32),
            pltpu.VMEM((CHUNK, 128), jnp.float32),
            pltpu.VMEM_SHARED((SN, 128), jnp.float32),
        ],
    )


TCB = 512


def _expand(a, b):
    return jnp.broadcast_to(a[:, None, :], (32, 32, b)).reshape(32 * 32, b)


def _fold(p, b):
    for h in (512, 256, 128, 64, 32):
        p = p[:h] + p[h:2 * h]
    return p


def _tc_compute_body(x_ref, ea_ref, esh_ref, w1t_ref, w2t_ref, tp_ref):
    b = TCB
    xt = x_ref[...].T
    ht = jnp.maximum(jnp.dot(w1t_ref[...], ea_ref[...],
                             preferred_element_type=jnp.float32), 0.0)
    hta = jnp.concatenate([ht, jnp.ones((1, b), jnp.float32)], axis=0)
    wt = jnp.dot(w2t_ref[...], hta.astype(jnp.bfloat16),
                 preferred_element_type=jnp.float32)

    s1 = xt[0:32]
    v1 = [xt[32 + 32 * m:64 + 32 * m] for m in range(3)]
    s2 = esh_ref[0:1]
    v2 = [esh_ref[1 + m:2 + m] for m in range(3)]

    d = v1[0] * v2[0] + v1[1] * v2[1] + v1[2] * v2[2]

    def contract(base, coef):
        acc = wt[base:base + 32] * coef[0:1]
        for u in range(1, 32):
            acc = acc + wt[base + 32 * u:base + 32 * (u + 1)] * coef[u:u + 1]
        return acc

    c1 = contract(0, s1)
    c2 = contract(1024, s1)
    c3 = [contract(2048, v1[m]) for m in range(3)]
    c4 = contract(3072, d)

    out0 = A0 * (c1 * s2 + ISQ3 * c4)
    o1 = [A1 * ISQ3 * (c2 * v2[m] + c3[m] * s2) for m in range(3)]
    tpt = jnp.concatenate([out0] + o1, axis=0)
    tp_ref[...] = tpt.T


def _tc_compute(x, eat, esht, w1t, w2t):
    grid = (EP // TCB,)
    return pl.pallas_call(
        _tc_compute_body,
        grid=grid,
        in_specs=[
            pl.BlockSpec((TCB, 128), lambda i: (i, 0)),
            pl.BlockSpec((17, TCB), lambda i: (0, i)),
            pl.BlockSpec((4, TCB), lambda i: (0, i)),
            pl.BlockSpec((64, 17), lambda i: (0, 0)),
            pl.BlockSpec((4096, 65), lambda i: (0, 0)),
        ],
        out_specs=pl.BlockSpec((TCB, 128), lambda i: (i, 0)),
        out_shape=jax.ShapeDtypeStruct((EP, 128), jnp.float32),
        compiler_params=pltpu.CompilerParams(
            dimension_semantics=("arbitrary",)),
    )(x, eat, esht, w1t, w2t)


FNB = 1000


def _tc_final_body(s0_ref, s1_ref, c0_ref, c1_ref, nap_ref, m_ref, out_ref):
    cnt = c0_ref[:, 0:1] + c1_ref[:, 0:1]
    denom = jnp.maximum(cnt, 1.0)
    res = (s0_ref[...] + s1_ref[...]) / denom + nap_ref[...]
    out_ref[...] = jnp.dot(res, m_ref[...], preferred_element_type=jnp.float32)


def _tc_finalize(s0, s1, c0, c1, nap, mperm):
    grid = (N_NODES // FNB,)
    return pl.pallas_call(
        _tc_final_body,
        grid=grid,
        in_specs=[
            pl.BlockSpec((FNB, 128), lambda i: (i, 0)),
            pl.BlockSpec((FNB, 128), lambda i: (i, 0)),
            pl.BlockSpec((FNB, 128), lambda i: (i, 0)),
            pl.BlockSpec((FNB, 128), lambda i: (i, 0)),
            pl.BlockSpec((FNB, 128), lambda i: (i, 0)),
            pl.BlockSpec((128, 128), lambda i: (0, 0)),
        ],
        out_specs=pl.BlockSpec((FNB, 128), lambda i: (i, 0)),
        out_shape=jax.ShapeDtypeStruct((N_NODES, 128), jnp.float32),
        compiler_params=pltpu.CompilerParams(
            dimension_semantics=("arbitrary",)),
    )(s0, s1, c0, c1, nap, mperm)


def kernel(node_attr, edge_index, edge_attr, edge_sh, W1, b1, W2, b2):
    f32 = jnp.float32
    nap = node_attr[:, _PERM]
    edge_dst = edge_index[0]
    edge_src = edge_index[1]
    pad = EP - N_EDGES

    dst_pad = jnp.concatenate(
        [edge_dst, jnp.zeros((pad,), jnp.int32)]).reshape(NW, CHUNKS, CHUNK)
    src_pad = jnp.concatenate(
        [edge_src, jnp.full((pad,), N_NODES, jnp.int32)]
    ).reshape(EP // CHUNK, CHUNK)

    eat = jnp.pad(
        jnp.concatenate([edge_attr.T, jnp.ones((1, N_EDGES), f32)], axis=0),
        ((0, 0), (0, pad)))
    esht = jnp.pad(edge_sh.T, ((0, 0), (0, pad)))
    w1t = jnp.concatenate([W1, b1[None, :]], axis=0).T
    w2t = jnp.concatenate([W2, b2[None, :]], axis=0).T.astype(jnp.bfloat16)

    zs = jnp.zeros((SN, 128), f32)
    ones = jnp.ones((CHUNK, 128), f32)
    cnts = _sc_scatter_cnts()(src_pad, zs, ones)
    x = _sc_gather()(nap, dst_pad)
    tp = _tc_compute(x, eat, esht, w1t, w2t)
    sums = _sc_scatter_sums()(tp, src_pad, zs)

    mperm = jnp.zeros((128, 128), f32).at[np.arange(128), _PERM].set(1.0)

    return _tc_finalize(sums[0, :N_NODES], sums[1, :N_NODES],
                        cnts[0, :N_NODES], cnts[1, :N_NODES], nap, mperm)


# speedup vs baseline: 6.6153x; 1.0852x over previous
import functools

import jax
import jax.numpy as jnp
import numpy as np
from jax import lax
from jax.experimental import pallas as pl
from jax.experimental.pallas import tpu as pltpu
from jax.experimental.pallas import tpu_sc as plsc

N_NODES = 10000
N_EDGES = 160000
MUL = 32

NW = 32
NBUF = 4
CHUNK = 128
CHUNKS = 40
EP = NW * CHUNKS * CHUNK
SN = 10112
STRIPE = SN // 16

ISQ3 = float(1.0 / np.sqrt(3.0))
A0 = float(np.sqrt(1.0 / 64.0))
A1 = float(np.sqrt(3.0 / 64.0))

@functools.cache
def _mesh():
    return plsc.VectorSubcoreMesh(core_axis_name="c", subcore_axis_name="s",
                                  num_cores=2, num_subcores=16)

_PERM = np.concatenate([np.arange(32)] + [32 + 3 * np.arange(32) + m for m in range(3)])


def _gather_body(nap_hbm, idx_hbm, out_hbm, idx_vm, rows_v, sem0, sem1):
    c = lax.axis_index("c")
    s = lax.axis_index("s")
    wid = s * 2 + c
    sems = (sem0, sem1)

    pltpu.sync_copy(idx_hbm.at[wid], idx_vm)

    def body(g, carry):
        j0 = NBUF * g
        gd = [pltpu.async_copy(nap_hbm.at[idx_vm.at[j0 + b]], rows_v.at[b],
                               sems[0]) for b in range(NBUF)]
        for d in gd:
            d.wait()
        wd = [pltpu.async_copy(
            rows_v.at[b],
            out_hbm.at[pl.ds((wid * CHUNKS + j0 + b) * CHUNK, CHUNK)],
            sems[1]) for b in range(NBUF)]
        for d in wd:
            d.wait()
        return carry

    lax.fori_loop(0, CHUNKS // NBUF, body, 0)


@functools.cache
def _sc_gather():
    return pl.kernel(
        _gather_body,
        out_type=jax.ShapeDtypeStruct((EP, 128), jnp.float32),
        mesh=_mesh(),
        scratch_types=[
            pltpu.VMEM((CHUNKS, CHUNK), jnp.int32),
            pltpu.VMEM((NBUF, CHUNK, 128), jnp.float32),
            pltpu.SemaphoreType.DMA,
            pltpu.SemaphoreType.DMA,
        ],
    )


def _scatter_sums_body(tp_hbm, src_hbm, zs_hbm, sums_out,
                       i0, i1, r0, r1, acc_sh, sem_i, sem_r):
    c = lax.axis_index("c")
    s = lax.axis_index("s")
    wid = s * 2 + c
    idxs = (i0, i1)
    rows = (r0, r1)

    pltpu.sync_copy(zs_hbm.at[pl.ds(s * STRIPE, STRIPE)],
                    acc_sh.at[pl.ds(s * STRIPE, STRIPE)])
    plsc.subcore_barrier()

    def body(g, carry):
        j0 = 2 * g
        ld = []
        for b in range(2):
            r = wid * CHUNKS + j0 + b
            ld.append(pltpu.async_copy(src_hbm.at[r], idxs[b], sem_i))
            ld.append(pltpu.async_copy(tp_hbm.at[pl.ds(r * CHUNK, CHUNK)],
                                       rows[b], sem_r))
        for b in range(2):
            ld[2 * b].wait()
            ld[2 * b + 1].wait()
            pltpu.sync_copy(rows[b], acc_sh.at[idxs[b]], add=True)
        return carry

    lax.fori_loop(0, CHUNKS // 2, body, 0)
    plsc.subcore_barrier()
    pltpu.sync_copy(acc_sh.at[pl.ds(s * STRIPE, STRIPE)],
                    sums_out.at[c, pl.ds(s * STRIPE, STRIPE)])


def _scatter_cnts_body(src_hbm, zs_hbm, ones_hbm, cnts_out,
                       i0, ones_v, acc_sh):
    c = lax.axis_index("c")
    s = lax.axis_index("s")
    wid = s * 2 + c
    pltpu.sync_copy(ones_hbm, ones_v)
    pltpu.sync_copy(zs_hbm.at[pl.ds(s * STRIPE, STRIPE)],
                    acc_sh.at[pl.ds(s * STRIPE, STRIPE)])
    plsc.subcore_barrier()

    def body(j, carry):
        r = wid * CHUNKS + j
        pltpu.sync_copy(src_hbm.at[r], i0)
        pltpu.sync_copy(ones_v, acc_sh.at[i0], add=True)
        return carry

    lax.fori_loop(0, CHUNKS, body, 0)
    plsc.subcore_barrier()
    pltpu.sync_copy(acc_sh.at[pl.ds(s * STRIPE, STRIPE)],
                    cnts_out.at[c, pl.ds(s * STRIPE, STRIPE)])


@functools.cache
def _sc_scatter_sums():
    return pl.kernel(
        _scatter_sums_body,
        out_type=jax.ShapeDtypeStruct((2, SN, 128), jnp.float32),
        mesh=_mesh(),
        scratch_types=[
            pltpu.VMEM((CHUNK,), jnp.int32),
            pltpu.VMEM((CHUNK,), jnp.int32),
            pltpu.VMEM((CHUNK, 128), jnp.float32),
            pltpu.VMEM((CHUNK, 128), jnp.float32),
            pltpu.VMEM_SHARED((SN, 128), jnp.float32),
            pltpu.SemaphoreType.DMA,
            pltpu.SemaphoreType.DMA,
        ],
    )


@functools.cache
def _sc_scatter_cnts():
    return pl.kernel(
        _scatter_cnts_body,
        out_type=jax.ShapeDtypeStruct((2, SN, 128), jnp.float32),
        mesh=_mesh(),
        scratch_types=[
            pltpu.VMEM((CHUNK,), jnp.int32),
            pltpu.VMEM((CHUNK, 128), jnp.float32),
            pltpu.VMEM_SHARED((SN, 128), jnp.float32),
        ],
    )


TCB = 1024


def _expand(a, b):
    return jnp.broadcast_to(a[:, None, :], (32, 32, b)).reshape(32 * 32, b)


def _fold(p, b):
    for h in (512, 256, 128, 64, 32):
        p = p[:h] + p[h:2 * h]
    return p


def _tc_compute_body(x_ref, ea_ref, esh_ref, w1t_ref, w2t_ref, tp_ref):
    b = TCB
    xt = x_ref[...].T
    ht = jnp.maximum(jnp.dot(w1t_ref[...], ea_ref[...],
                             preferred_element_type=jnp.float32), 0.0)
    hta = jnp.concatenate([ht, jnp.ones((1, b), jnp.float32)], axis=0)
    wt = jnp.dot(w2t_ref[...], hta.astype(jnp.bfloat16),
                 preferred_element_type=jnp.float32)

    s1 = xt[0:32]
    v1 = [xt[32 + 32 * m:64 + 32 * m] for m in range(3)]
    s2 = esh_ref[0:1]
    v2 = [esh_ref[1 + m:2 + m] for m in range(3)]

    d = v1[0] * v2[0] + v1[1] * v2[1] + v1[2] * v2[2]

    def contract(base, coef):
        acc = wt[base:base + 32] * coef[0:1]
        for u in range(1, 32):
            acc = acc + wt[base + 32 * u:base + 32 * (u + 1)] * coef[u:u + 1]
        return acc

    c1 = contract(0, s1)
    c2 = contract(1024, s1)
    c3 = [contract(2048, v1[m]) for m in range(3)]
    c4 = contract(3072, d)

    out0 = A0 * (c1 * s2 + ISQ3 * c4)
    o1 = [A1 * ISQ3 * (c2 * v2[m] + c3[m] * s2) for m in range(3)]
    tpt = jnp.concatenate([out0] + o1, axis=0)
    tp_ref[...] = tpt.T


def _tc_compute(x, eat, esht, w1t, w2t):
    grid = (EP // TCB,)
    return pl.pallas_call(
        _tc_compute_body,
        grid=grid,
        in_specs=[
            pl.BlockSpec((TCB, 128), lambda i: (i, 0)),
            pl.BlockSpec((17, TCB), lambda i: (0, i)),
            pl.BlockSpec((4, TCB), lambda i: (0, i)),
            pl.BlockSpec((64, 17), lambda i: (0, 0)),
            pl.BlockSpec((4096, 65), lambda i: (0, 0)),
        ],
        out_specs=pl.BlockSpec((TCB, 128), lambda i: (i, 0)),
        out_shape=jax.ShapeDtypeStruct((EP, 128), jnp.float32),
        compiler_params=pltpu.CompilerParams(
            dimension_semantics=("arbitrary",)),
    )(x, eat, esht, w1t, w2t)


FNB = 1000


def _tc_final_body(s0_ref, s1_ref, c0_ref, c1_ref, nap_ref, m_ref, out_ref):
    cnt = c0_ref[:, 0:1] + c1_ref[:, 0:1]
    denom = jnp.maximum(cnt, 1.0)
    res = (s0_ref[...] + s1_ref[...]) / denom + nap_ref[...]
    out_ref[...] = jnp.dot(res, m_ref[...], preferred_element_type=jnp.float32)


def _tc_finalize(s0, s1, c0, c1, nap, mperm):
    grid = (N_NODES // FNB,)
    return pl.pallas_call(
        _tc_final_body,
        grid=grid,
        in_specs=[
            pl.BlockSpec((FNB, 128), lambda i: (i, 0)),
            pl.BlockSpec((FNB, 128), lambda i: (i, 0)),
            pl.BlockSpec((FNB, 128), lambda i: (i, 0)),
            pl.BlockSpec((FNB, 128), lambda i: (i, 0)),
            pl.BlockSpec((FNB, 128), lambda i: (i, 0)),
            pl.BlockSpec((128, 128), lambda i: (0, 0)),
        ],
        out_specs=pl.BlockSpec((FNB, 128), lambda i: (i, 0)),
        out_shape=jax.ShapeDtypeStruct((N_NODES, 128), jnp.float32),
        compiler_params=pltpu.CompilerParams(
            dimension_semantics=("arbitrary",)),
    )(s0, s1, c0, c1, nap, mperm)


def kernel(node_attr, edge_index, edge_attr, edge_sh, W1, b1, W2, b2):
    f32 = jnp.float32
    nap = node_attr[:, _PERM]
    edge_dst = edge_index[0]
    edge_src = edge_index[1]
    pad = EP - N_EDGES

    dst_pad = jnp.concatenate(
        [edge_dst, jnp.zeros((pad,), jnp.int32)]).reshape(NW, CHUNKS, CHUNK)
    src_pad = jnp.concatenate(
        [edge_src, jnp.full((pad,), N_NODES, jnp.int32)]
    ).reshape(EP // CHUNK, CHUNK)

    eat = jnp.pad(
        jnp.concatenate([edge_attr.T, jnp.ones((1, N_EDGES), f32)], axis=0),
        ((0, 0), (0, pad)))
    esht = jnp.pad(edge_sh.T, ((0, 0), (0, pad)))
    w1t = jnp.concatenate([W1, b1[None, :]], axis=0).T
    w2t = jnp.concatenate([W2, b2[None, :]], axis=0).T.astype(jnp.bfloat16)

    zs = jnp.zeros((SN, 128), f32)
    ones = jnp.ones((CHUNK, 128), f32)
    cnts = _sc_scatter_cnts()(src_pad, zs, ones)
    x = _sc_gather()(nap, dst_pad)
    tp = _tc_compute(x, eat, esht, w1t, w2t)
    sums = _sc_scatter_sums()(tp, src_pad, zs)

    mperm = jnp.zeros((128, 128), f32).at[np.arange(128), _PERM].set(1.0)

    return _tc_finalize(sums[0, :N_NODES], sums[1, :N_NODES],
                        cnts[0, :N_NODES], cnts[1, :N_NODES], nap, mperm)
